# baseline (speedup 1.0000x reference)
import functools

import jax
import jax.numpy as jnp
from jax import lax
from jax.experimental import pallas as pl
from jax.experimental.pallas import tpu as pltpu
from jax.experimental.pallas import tpu_sc as plsc

N = 10000
D = 256
H = 256
E = 160000
HALF = 128

NS = 16
B = 128
SUBS = 4
NCH = 80
NSEC = 2
SECN = NCH // NSEC
EPAD = NS * NCH * B
RPT = 624
REM = N - NS * RPT
MM_R = 1000


def _mm_body(x_ref, w_ref, lna, lnb, c1a, c1b, c2a, c2b):
    y = jnp.dot(x_ref[...], w_ref[...], preferred_element_type=jnp.float32)
    lna[...] = y[:, 0:128]
    lnb[...] = y[:, 128:256]
    c1a[...] = y[:, 256:384]
    c1b[...] = y[:, 384:512]
    c2a[...] = y[:, 512:640]
    c2b[...] = y[:, 640:768]


def _matmul3(x, wcat):
    return pl.pallas_call(
        _mm_body,
        grid=(N // MM_R,),
        in_specs=[
            pl.BlockSpec((MM_R, D), lambda i: (i, 0)),
            pl.BlockSpec((D, 3 * H), lambda i: (0, 0)),
        ],
        out_specs=[pl.BlockSpec((MM_R, HALF), lambda i: (i, 0))] * 6,
        out_shape=[jax.ShapeDtypeStruct((N, HALF), jnp.float32)] * 6,
    )(x, wcat)


def _sc_body(lna, lnb, t1a, t1b, t2a, t2b,
             s1, d1, e1, s2, d2, e2,
             out, acc, srcv, dstv, eav, rows_a, rows_b, sem_a, sem_b):
    c = lax.axis_index("c")
    s = lax.axis_index("s")
    r0 = s * RPT

    def half(ln, t1, t2, col0):
        pltpu.sync_copy(ln.at[pl.ds(r0, RPT)], acc.at[pl.ds(r0, RPT)])

        @pl.when(s == NS - 1)
        def _init_rem():
            pltpu.sync_copy(ln.at[pl.ds(NS * RPT, REM)],
                            acc.at[pl.ds(NS * RPT, REM)])

        plsc.subcore_barrier()

        def scale(rows, i):
            def group(g, carry):
                ea16 = eav[i, pl.ds(g * 16, 16)]
                for l in range(16):
                    a = ea16[l]
                    e = g * 16 + l
                    for j in range(HALF // 16):
                        sl = pl.ds(j * 16, 16)
                        rows[e, sl] = rows[e, sl] * a
                return carry

            lax.fori_loop(0, B // 16, group, 0)

        for (t, sv, dv, ev) in ((t1, s1, d1, e1), (t2, s2, d2, e2)):
            for sec in range(NSEC):
                pltpu.sync_copy(sv.at[s, pl.ds(sec * SECN, SECN)], srcv)
                pltpu.sync_copy(dv.at[s, pl.ds(sec * SECN, SECN)], dstv)
                pltpu.sync_copy(ev.at[s, pl.ds(sec * SECN, SECN)], eav)

                def issue_gather(i, rows, sem):
                    for q in range(SUBS):
                        sl = pl.ds(q * (B // SUBS), B // SUBS)
                        pltpu.async_copy(t.at[srcv.at[i, sl]],
                                         rows.at[sl], sem)

                def drain_gather(rows, sem):
                    for q in range(SUBS):
                        sl = pl.ds(q * (B // SUBS), B // SUBS)
                        pltpu.make_async_copy(t.at[pl.ds(0, B // SUBS)],
                                              rows.at[sl], sem).wait()

                issue_gather(0, rows_a, sem_a)

                def pair(p, carry):
                    ia = 2 * p
                    ib = 2 * p + 1
                    drain_gather(rows_a, sem_a)
                    issue_gather(ib, rows_b, sem_b)
                    scale(rows_a, ia)
                    pltpu.sync_copy(rows_a, acc.at[dstv.at[ia]], add=True)
                    drain_gather(rows_b, sem_b)

                    @pl.when(ib + 1 < SECN)
                    def _next():
                        issue_gather(ib + 1, rows_a, sem_a)

                    scale(rows_b, ib)
                    pltpu.sync_copy(rows_b, acc.at[dstv.at[ib]], add=True)
                    return carry

                lax.fori_loop(0, SECN // 2, pair, 0)

        plsc.subcore_barrier()
        pltpu.sync_copy(acc.at[pl.ds(r0, RPT)],
                        out.at[pl.ds(r0, RPT), pl.ds(col0, HALF)])

        @pl.when(s == NS - 1)
        def _out_rem():
            pltpu.sync_copy(acc.at[pl.ds(NS * RPT, REM)],
                            out.at[pl.ds(NS * RPT, REM), pl.ds(col0, HALF)])

    @pl.when(c == 0)
    def _half0():
        half(lna, t1a, t2a, 0)

    @pl.when(c == 1)
    def _half1():
        half(lnb, t1b, t2b, HALF)


@functools.cache
def _sc_block():
    return pl.kernel(
        _sc_body,
        out_type=jax.ShapeDtypeStruct((N, H), jnp.float32),
        mesh=plsc.VectorSubcoreMesh(core_axis_name="c", subcore_axis_name="s"),
        scratch_types=[
            pltpu.VMEM_SHARED((N, HALF), jnp.float32),
            pltpu.VMEM((SECN, B), jnp.int32),
            pltpu.VMEM((SECN, B), jnp.int32),
            pltpu.VMEM((SECN, B), jnp.float32),
            pltpu.VMEM((B, HALF), jnp.float32),
            pltpu.VMEM((B, HALF), jnp.float32),
            pltpu.SemaphoreType.DMA,
            pltpu.SemaphoreType.DMA,
        ],
    )


def _block(x, wcat, s1, d1, e1, s2, d2, e2):
    lna, lnb, c1a, c1b, c2a, c2b = _matmul3(x, wcat)
    return _sc_block()(lna, lnb, c1a, c1b, c2a, c2b, s1, d1, e1, s2, d2, e2)


def kernel(x, edge_index, edge_attr, edge_index2, edge_attr2, batch,
           W0_ln, W0_c1, W0_c2, W1_ln, W1_c1, W1_c2):
    def _pad_i(v):
        return jnp.concatenate(
            [v.astype(jnp.int32), jnp.zeros((EPAD - E,), jnp.int32)]
        ).reshape(NS, NCH, B)

    def _pad_f(v):
        return jnp.concatenate(
            [v, jnp.zeros((EPAD - E,), jnp.float32)]
        ).reshape(NS, NCH, B)

    s1 = _pad_i(edge_index[0])
    d1 = _pad_i(edge_index[1])
    e1 = _pad_f(edge_attr)
    s2 = _pad_i(edge_index2[0])
    d2 = _pad_i(edge_index2[1])
    e2 = _pad_f(edge_attr2)

    wcat0 = jnp.concatenate([W0_ln, W0_c1, W0_c2], axis=1)
    wcat1 = jnp.concatenate([W1_ln, W1_c1, W1_c2], axis=1)

    h = _block(x, wcat0, s1, d1, e1, s2, d2, e2)
    return _block(h, wcat1, s1, d1, e1, s2, d2, e2)


# speedup vs baseline: 1.0340x; 1.0340x over previous
import functools

import jax
import jax.numpy as jnp
from jax import lax
from jax.experimental import pallas as pl
from jax.experimental.pallas import tpu as pltpu
from jax.experimental.pallas import tpu_sc as plsc

N = 10000
D = 256
H = 256
E = 160000
HALF = 128

NS = 16
B = 128
NCH = 79
EPAD = NS * NCH * B
RPT = 624
REM = N - NS * RPT
MM_R = 1000


def _mm_body(x_ref, w_ref, lna, lnb, c1a, c1b, c2a, c2b):
    y = jnp.dot(x_ref[...], w_ref[...], preferred_element_type=jnp.float32)
    lna[...] = y[:, 0:128]
    lnb[...] = y[:, 128:256]
    c1a[...] = y[:, 256:384]
    c1b[...] = y[:, 384:512]
    c2a[...] = y[:, 512:640]
    c2b[...] = y[:, 640:768]


def _matmul3(x, wcat):
    return pl.pallas_call(
        _mm_body,
        grid=(N // MM_R,),
        in_specs=[
            pl.BlockSpec((MM_R, D), lambda i: (i, 0)),
            pl.BlockSpec((D, 3 * H), lambda i: (0, 0)),
        ],
        out_specs=[pl.BlockSpec((MM_R, HALF), lambda i: (i, 0))] * 6,
        out_shape=[jax.ShapeDtypeStruct((N, HALF), jnp.float32)] * 6,
    )(x, wcat)


def _sc_body(lna, lnb, t1a, t1b, t2a, t2b,
             s1, d1, e1, s2, d2, e2,
             out, acc, srcv, dstv, eav, rows, sem):
    c = lax.axis_index("c")
    s = lax.axis_index("s")
    r0 = s * RPT

    def half(ln, t1, t2, col0):
        pltpu.sync_copy(ln.at[pl.ds(r0, RPT)], acc.at[pl.ds(r0, RPT)])

        @pl.when(s == NS - 1)
        def _init_rem():
            pltpu.sync_copy(ln.at[pl.ds(NS * RPT, REM)],
                            acc.at[pl.ds(NS * RPT, REM)])

        plsc.subcore_barrier()

        for (t, sv, dv, ev) in ((t1, s1, d1, e1), (t2, s2, d2, e2)):
            pltpu.sync_copy(sv.at[s], srcv)
            pltpu.sync_copy(dv.at[s], dstv)
            pltpu.sync_copy(ev.at[s], eav)

            def chunk(i, carry):
                pltpu.async_copy(t.at[srcv.at[i]], rows, sem).wait()

                def scale(g, carry2):
                    ea16 = eav[i, pl.ds(g * 16, 16)]
                    for l in range(16):
                        a = ea16[l]
                        e = g * 16 + l
                        for j in range(HALF // 16):
                            sl = pl.ds(j * 16, 16)
                            rows[e, sl] = rows[e, sl] * a
                    return carry2

                lax.fori_loop(0, B // 16, scale, 0)
                pltpu.sync_copy(rows, acc.at[dstv.at[i]], add=True)
                return carry

            lax.fori_loop(0, NCH, chunk, 0)

        plsc.subcore_barrier()
        pltpu.sync_copy(acc.at[pl.ds(r0, RPT)],
                        out.at[pl.ds(r0, RPT), pl.ds(col0, HALF)])

        @pl.when(s == NS - 1)
        def _out_rem():
            pltpu.sync_copy(acc.at[pl.ds(NS * RPT, REM)],
                            out.at[pl.ds(NS * RPT, REM), pl.ds(col0, HALF)])

    @pl.when(c == 0)
    def _half0():
        half(lna, t1a, t2a, 0)

    @pl.when(c == 1)
    def _half1():
        half(lnb, t1b, t2b, HALF)


@functools.cache
def _sc_block():
    return pl.kernel(
        _sc_body,
        out_type=jax.ShapeDtypeStruct((N, H), jnp.float32),
        mesh=plsc.VectorSubcoreMesh(core_axis_name="c", subcore_axis_name="s"),
        scratch_types=[
            pltpu.VMEM_SHARED((N, HALF), jnp.float32),
            pltpu.VMEM((NCH, B), jnp.int32),
            pltpu.VMEM((NCH, B), jnp.int32),
            pltpu.VMEM((NCH, B), jnp.float32),
            pltpu.VMEM((B, HALF), jnp.float32),
            pltpu.SemaphoreType.DMA,
        ],
    )


def _block(x, wcat, s1, d1, e1, s2, d2, e2):
    lna, lnb, c1a, c1b, c2a, c2b = _matmul3(x, wcat)
    return _sc_block()(lna, lnb, c1a, c1b, c2a, c2b, s1, d1, e1, s2, d2, e2)


def kernel(x, edge_index, edge_attr, edge_index2, edge_attr2, batch,
           W0_ln, W0_c1, W0_c2, W1_ln, W1_c1, W1_c2):
    def _pad_i(v):
        return jnp.concatenate(
            [v.astype(jnp.int32), jnp.zeros((EPAD - E,), jnp.int32)]
        ).reshape(NS, NCH, B)

    def _pad_f(v):
        return jnp.concatenate(
            [v, jnp.zeros((EPAD - E,), jnp.float32)]
        ).reshape(NS, NCH, B)

    s1 = _pad_i(edge_index[0])
    d1 = _pad_i(edge_index[1])
    e1 = _pad_f(edge_attr)
    s2 = _pad_i(edge_index2[0])
    d2 = _pad_i(edge_index2[1])
    e2 = _pad_f(edge_attr2)

    wcat0 = jnp.concatenate([W0_ln, W0_c1, W0_c2], axis=1)
    wcat1 = jnp.concatenate([W1_ln, W1_c1, W1_c2], axis=1)

    h = _block(x, wcat0, s1, d1, e1, s2, d2, e2)
    return _block(h, wcat1, s1, d1, e1, s2, d2, e2)
